# baseline (speedup 1.0000x reference)
import functools

import jax
import jax.numpy as jnp
from jax import lax
from jax.experimental import pallas as pl
from jax.experimental.pallas import tpu as pltpu
from jax.experimental.pallas import tpu_sc as plsc

NUM_SITES = 1000
EMBED_DIM = 16
BATCH = 16384
HIST = 200

NC, NS = 2, 16
NW = NC * NS
B_TOTAL = BATCH * HIST
B_PER_W = B_TOTAL // NW
CHUNK = 3200
NCHUNK = B_PER_W // CHUNK
NBUF = 2
NITER = NCHUNK // NBUF


def _build():
    mesh = plsc.VectorSubcoreMesh(core_axis_name="c", subcore_axis_name="s")

    @functools.partial(
        pl.kernel,
        out_type=jax.ShapeDtypeStruct((B_TOTAL, EMBED_DIM), jnp.float32),
        mesh=mesh,
        scratch_types=[
            pltpu.VMEM((NBUF, CHUNK), jnp.int32),
            pltpu.VMEM((NBUF, CHUNK, EMBED_DIM), jnp.float32),
            pltpu.SemaphoreType.DMA((NBUF,)),
            pltpu.SemaphoreType.DMA((NBUF,)),
            pltpu.SemaphoreType.DMA((NBUF,)),
        ],
        compiler_params=pltpu.CompilerParams(use_tc_tiling_on_sc=False),
    )
    def emb(idx_hbm, table_hbm, out_hbm, idx_v, rows_v, isem, gsem, osem):
        wid = lax.axis_index("s") * NC + lax.axis_index("c")
        base = wid * B_PER_W

        def idx_cp(c, b):
            return pltpu.make_async_copy(
                idx_hbm.at[pl.ds(base + c * CHUNK, CHUNK)], idx_v.at[b],
                isem.at[b])

        def gat_cp(b):
            return pltpu.make_async_copy(
                table_hbm.at[idx_v.at[b]], rows_v.at[b], gsem.at[b])

        def out_cp(c, b):
            return pltpu.make_async_copy(
                rows_v.at[b], out_hbm.at[pl.ds(base + c * CHUNK, CHUNK)],
                osem.at[b])

        for b in range(NBUF):
            idx_cp(b, b).start()

        def body(g, carry):
            for b in range(NBUF):
                c = g * NBUF + b
                idx_cp(c, b).wait()

                @pl.when(g > 0)
                def _():
                    out_cp(c, b).wait()

                gather = gat_cp(b)
                gather.start()
                gather.wait()

                @pl.when(g < NITER - 1)
                def _():
                    idx_cp(c + NBUF, b).start()

                out_cp(c, b).start()
            return carry

        lax.fori_loop(0, NITER, body, 0)
        for b in range(NBUF):
            out_cp(NCHUNK - NBUF + b, b).wait()

    return emb


_emb = _build()


def kernel(sites, input, target, site_embeddings):
    idx = sites.reshape(B_TOTAL)
    out = _emb(idx, site_embeddings)
    return out.reshape(BATCH, HIST, EMBED_DIM)


# speedup vs baseline: 1.0033x; 1.0033x over previous
import functools

import jax
import jax.numpy as jnp
from jax import lax
from jax.experimental import pallas as pl
from jax.experimental.pallas import tpu as pltpu
from jax.experimental.pallas import tpu_sc as plsc

NUM_SITES = 1000
EMBED_DIM = 16
BATCH = 16384
HIST = 200

NC, NS = 2, 16
NW = NC * NS
ROWS_PER_W = BATCH // NW
RCHUNK = 16
CHUNK = RCHUNK * HIST
NCHUNK = ROWS_PER_W // RCHUNK
NBUF = 2
NITER = NCHUNK // NBUF


def _build():
    mesh = plsc.VectorSubcoreMesh(core_axis_name="c", subcore_axis_name="s")

    @functools.partial(
        pl.kernel,
        out_type=jax.ShapeDtypeStruct((BATCH, HIST, EMBED_DIM), jnp.float32),
        mesh=mesh,
        scratch_types=[
            pltpu.VMEM((NBUF, CHUNK), jnp.int32),
            pltpu.VMEM((NBUF, RCHUNK, HIST, EMBED_DIM), jnp.float32),
            pltpu.SemaphoreType.DMA((NBUF,)),
            pltpu.SemaphoreType.DMA((NBUF,)),
            pltpu.SemaphoreType.DMA((NBUF,)),
        ],
        compiler_params=pltpu.CompilerParams(use_tc_tiling_on_sc=False),
    )
    def emb(idx_hbm, table_hbm, out_hbm, idx_v, rows_v, isem, gsem, osem):
        wid = lax.axis_index("s") * NC + lax.axis_index("c")
        rbase = wid * ROWS_PER_W

        def idx_cp(c, b):
            off = (rbase + c * RCHUNK) * HIST
            return pltpu.make_async_copy(
                idx_hbm.at[pl.ds(off, CHUNK)], idx_v.at[b], isem.at[b])

        def gat_cp(b, j):
            return pltpu.make_async_copy(
                table_hbm.at[idx_v.at[b, pl.ds(j * HIST, HIST)]],
                rows_v.at[b, j], gsem.at[b])

        def out_cp(c, b):
            return pltpu.make_async_copy(
                rows_v.at[b], out_hbm.at[pl.ds(rbase + c * RCHUNK, RCHUNK)],
                osem.at[b])

        for b in range(NBUF):
            idx_cp(b, b).start()

        def body(g, carry):
            for b in range(NBUF):
                c = g * NBUF + b
                idx_cp(c, b).wait()

                @pl.when(g > 0)
                def _():
                    out_cp(c, b).wait()

                for j in range(RCHUNK):
                    gat_cp(b, j).start()
                for j in range(RCHUNK):
                    gat_cp(b, j).wait()

                @pl.when(g < NITER - 1)
                def _():
                    idx_cp(c + NBUF, b).start()

                out_cp(c, b).start()
            return carry

        lax.fori_loop(0, NITER, body, 0)
        for b in range(NBUF):
            out_cp(NCHUNK - NBUF + b, b).wait()

    return emb


_emb = _build()


def kernel(sites, input, target, site_embeddings):
    idx = sites.reshape(BATCH * HIST)
    return _emb(idx, site_embeddings)


# speedup vs baseline: 2.2716x; 2.2642x over previous
import functools

import jax
import jax.numpy as jnp
from jax import lax
from jax.experimental import pallas as pl
from jax.experimental.pallas import tpu as pltpu
from jax.experimental.pallas import tpu_sc as plsc

NUM_SITES = 1000
EMBED_DIM = 16
BATCH = 16384
HIST = 200

NC, NS = 2, 16
NW = NC * NS
B_PER_W = BATCH // NW
BBLK = 128
NBLK = B_PER_W // BBLK
IDXC = BBLK * HIST
NBUF = 2


def _build():
    mesh = plsc.VectorSubcoreMesh(core_axis_name="c", subcore_axis_name="s")

    @functools.partial(
        pl.kernel,
        out_type=jax.ShapeDtypeStruct((HIST, EMBED_DIM, BATCH), jnp.float32),
        mesh=mesh,
        scratch_types=[
            pltpu.VMEM((NUM_SITES * EMBED_DIM,), jnp.float32),
            pltpu.VMEM((IDXC,), jnp.int32),
            pltpu.VMEM((NBUF, EMBED_DIM, BBLK), jnp.float32),
            pltpu.SemaphoreType.DMA((NBUF,)),
        ],
        compiler_params=pltpu.CompilerParams(
            use_tc_tiling_on_sc=True, needs_layout_passes=False),
    )
    def emb(idx_hbm, table_hbm, out_hbm, table_v, idx_v, stage_v, osem):
        wid = lax.axis_index("s") * NC + lax.axis_index("c")
        bbase = wid * B_PER_W
        pltpu.sync_copy(table_hbm, table_v)
        lane = lax.iota(jnp.int32, 16)
        lane_h = lane * HIST

        for blk in range(NBLK):
            b0 = bbase + blk * BBLK
            pltpu.sync_copy(idx_hbm.at[pl.ds(b0 * HIST, IDXC)], idx_v)

            def out_cp(h, p):
                return pltpu.make_async_copy(
                    stage_v.at[p],
                    out_hbm.at[h, slice(None), pl.ds(b0, BBLK)],
                    osem.at[p])

            def hbody(hh, carry):
                for p in range(NBUF):
                    h = hh * NBUF + p

                    @pl.when(jnp.logical_or(hh > 0, blk > 0))
                    def _():
                        out_cp(h, p).wait()

                    for g in range(BBLK // 16):
                        sid = plsc.load_gather(idx_v, [lane_h + (g * 16 * HIST + h)])
                        row = sid * EMBED_DIM
                        for d in range(EMBED_DIM):
                            vals = plsc.load_gather(table_v, [row + d])
                            stage_v[p, d, pl.ds(g * 16, 16)] = vals
                    out_cp(h, p).start()
                return carry

            lax.fori_loop(0, HIST // NBUF, hbody, 0)

        for p in range(NBUF):
            pltpu.make_async_copy(
                stage_v.at[p],
                out_hbm.at[HIST - NBUF + p, slice(None),
                           pl.ds(bbase + (NBLK - 1) * BBLK, BBLK)],
                osem.at[p]).wait()

    return emb


_emb = _build()


def kernel(sites, input, target, site_embeddings):
    idx = sites.reshape(BATCH * HIST)
    table_flat = site_embeddings.reshape(NUM_SITES * EMBED_DIM)
    out_t = _emb(idx, table_flat)
    return jnp.transpose(out_t, (2, 0, 1))


# speedup vs baseline: 4.9470x; 2.1777x over previous
import functools

import jax
import jax.numpy as jnp
from jax import lax
from jax.experimental import pallas as pl
from jax.experimental.pallas import tpu as pltpu
from jax.experimental.pallas import tpu_sc as plsc

NUM_SITES = 1000
EMBED_DIM = 16
BATCH = 16384
HIST = 200

NC, NS = 2, 16
NW = NC * NS
B_PER_W = BATCH // NW
BBLK = 128
NBLK = B_PER_W // BBLK
IDXC = BBLK * HIST
NBUF = 2


def _build():
    mesh = plsc.VectorSubcoreMesh(core_axis_name="c", subcore_axis_name="s")

    @functools.partial(
        pl.kernel,
        out_type=jax.ShapeDtypeStruct((HIST, EMBED_DIM, BATCH), jnp.float32),
        mesh=mesh,
        scratch_types=[
            pltpu.VMEM((NUM_SITES * EMBED_DIM,), jnp.float32),
            pltpu.VMEM((IDXC,), jnp.int32),
            pltpu.VMEM((NBUF, EMBED_DIM, BBLK), jnp.float32),
            pltpu.SemaphoreType.DMA((NBUF,)),
        ],
        compiler_params=pltpu.CompilerParams(
            use_tc_tiling_on_sc=True, needs_layout_passes=False,
            disable_bounds_checks=True),
    )
    def emb(idx_hbm, table_hbm, out_hbm, table_v, idx_v, stage_v, osem):
        wid = lax.axis_index("s") * NC + lax.axis_index("c")
        bbase = wid * B_PER_W
        pltpu.sync_copy(table_hbm, table_v)
        lane = lax.iota(jnp.int32, 16)
        lane_h = lane * HIST

        for blk in range(NBLK):
            b0 = bbase + blk * BBLK
            pltpu.sync_copy(idx_hbm.at[pl.ds(b0 * HIST, IDXC)], idx_v)

            def out_cp(h, p):
                return pltpu.make_async_copy(
                    stage_v.at[p],
                    out_hbm.at[h, slice(None), pl.ds(b0, BBLK)],
                    osem.at[p])

            def hbody(hh, carry):
                for p in range(NBUF):
                    h = hh * NBUF + p

                    @pl.when(jnp.logical_or(hh > 0, blk > 0))
                    def _():
                        out_cp(h, p).wait()

                    for g in range(BBLK // 16):
                        sid = plsc.load_gather(idx_v, [lane_h + (g * 16 * HIST + h)])
                        row = sid * EMBED_DIM
                        vals = [plsc.load_gather(table_v, [row + d])
                                for d in range(EMBED_DIM)]
                        for d in range(EMBED_DIM):
                            stage_v[p, d, pl.ds(g * 16, 16)] = vals[d]
                    out_cp(h, p).start()
                return carry

            lax.fori_loop(0, HIST // NBUF, hbody, 0)

        for p in range(NBUF):
            pltpu.make_async_copy(
                stage_v.at[p],
                out_hbm.at[HIST - NBUF + p, slice(None),
                           pl.ds(bbase + (NBLK - 1) * BBLK, BBLK)],
                osem.at[p]).wait()

    return emb


_emb = _build()


def kernel(sites, input, target, site_embeddings):
    idx = sites.reshape(BATCH * HIST)
    table_flat = site_embeddings.reshape(NUM_SITES * EMBED_DIM)
    out_t = _emb(idx, table_flat)
    return jnp.transpose(out_t, (2, 0, 1))


# speedup vs baseline: 4.9800x; 1.0067x over previous
import functools

import jax
import jax.numpy as jnp
from jax import lax
from jax.experimental import pallas as pl
from jax.experimental.pallas import tpu as pltpu
from jax.experimental.pallas import tpu_sc as plsc

NUM_SITES = 1000
EMBED_DIM = 16
BATCH = 16384
HIST = 200

NC, NS = 2, 16
NW = NC * NS
B_PER_W = BATCH // NW
BBLK = 128
NBLK = B_PER_W // BBLK
IDXC = BBLK * HIST
NBUF = 2


def _build():
    mesh = plsc.VectorSubcoreMesh(core_axis_name="c", subcore_axis_name="s")

    @functools.partial(
        pl.kernel,
        out_type=jax.ShapeDtypeStruct((HIST, EMBED_DIM, BATCH), jnp.float32),
        mesh=mesh,
        scratch_types=[
            pltpu.VMEM((NUM_SITES * EMBED_DIM,), jnp.float32),
            pltpu.VMEM((IDXC,), jnp.int32),
            pltpu.VMEM((NBUF, EMBED_DIM, BBLK), jnp.float32),
            pltpu.SemaphoreType.DMA((NBUF,)),
        ],
        compiler_params=pltpu.CompilerParams(
            use_tc_tiling_on_sc=True, needs_layout_passes=False,
            disable_bounds_checks=True),
    )
    def emb(idx_hbm, table_hbm, out_hbm, table_v, idx_v, stage_v, osem):
        wid = lax.axis_index("s") * NC + lax.axis_index("c")
        bbase = wid * B_PER_W
        pltpu.sync_copy(table_hbm, table_v)
        lane = lax.iota(jnp.int32, 16)
        lane_h = lane * HIST

        for blk in range(NBLK):
            b0 = bbase + blk * BBLK
            pltpu.sync_copy(idx_hbm.at[pl.ds(b0 * HIST, IDXC)], idx_v)

            def out_cp(h, p):
                return pltpu.make_async_copy(
                    stage_v.at[p],
                    out_hbm.at[h, slice(None), pl.ds(b0, BBLK)],
                    osem.at[p])

            def hbody(hh, carry):
                for p in range(NBUF):
                    h = hh * NBUF + p

                    @pl.when(jnp.logical_or(hh > 0, blk > 0))
                    def _():
                        out_cp(h, p).wait()

                    def gather_group(g):
                        sid = plsc.load_gather(
                            idx_v, [lane_h + (g * 16 * HIST + h)])
                        row = sid * EMBED_DIM
                        return [plsc.load_gather(table_v, [row + d])
                                for d in range(EMBED_DIM)]

                    def store_group(g, vals):
                        for d in range(EMBED_DIM):
                            stage_v[p, d, pl.ds(g * 16, 16)] = vals[d]

                    vals = gather_group(0)
                    for g in range(1, BBLK // 16):
                        nxt = gather_group(g)
                        store_group(g - 1, vals)
                        vals = nxt
                    store_group(BBLK // 16 - 1, vals)
                    out_cp(h, p).start()
                return carry

            lax.fori_loop(0, HIST // NBUF, hbody, 0)

        for p in range(NBUF):
            pltpu.make_async_copy(
                stage_v.at[p],
                out_hbm.at[HIST - NBUF + p, slice(None),
                           pl.ds(bbase + (NBLK - 1) * BBLK, BBLK)],
                osem.at[p]).wait()

    return emb


_emb = _build()


def kernel(sites, input, target, site_embeddings):
    idx = sites.reshape(BATCH * HIST)
    table_flat = site_embeddings.reshape(NUM_SITES * EMBED_DIM)
    out_t = _emb(idx, table_flat)
    return jnp.transpose(out_t, (2, 0, 1))
